# baseline (speedup 1.0000x reference)
import functools

import numpy as np
import jax
import jax.numpy as jnp
from jax.experimental import pallas as pl
from jax.experimental.pallas import tpu as pltpu

NUMANGLE = 180
NUMRHO = 184
R_PAD = 192
OUT_H = 128
OUT_W = 128
P = OUT_H * OUT_W

P_TILE = 2048
A_BLK = 12
K = A_BLK * R_PAD


def _rho_index_table(H, W, numangle, numrho):
    irho = float(int(np.sqrt(H * H + W * W) + 1)) / float(numrho - 1)
    angles = np.arange(numangle).astype(np.float64) * (np.pi / numangle)
    cosi = np.cos(angles) / irho
    sini = np.sin(angles) / irho
    xs = (np.arange(W) - W // 2).astype(np.float64)
    ys = (np.arange(H) - H // 2).astype(np.float64)
    r = np.round(
        cosi[:, None, None] * xs[None, None, :] + sini[:, None, None] * ys[None, :, None]
    ).astype(np.int32) + numrho // 2
    invalid = (r < 0) | (r >= numrho)
    r[invalid] = numrho
    return r.reshape(numangle // A_BLK, A_BLK, H * W)


N_A = NUMANGLE // A_BLK
N_P = P // P_TILE


def _gen_onehot(ridx_ref, oh_ref, base):
    iota = jax.lax.broadcasted_iota(jnp.int32, (R_PAD, P_TILE), 0)
    for j in range(A_BLK):
        oh_ref[pl.ds(base + j * R_PAD, R_PAD), :] = (
            iota == ridx_ref[0, j, :][None, :]
        ).astype(jnp.bfloat16)


def _idht_block(ridx_cur_ref, ridx_nxt_ref, acc_ref, out_ref, oh_ref):
    p = pl.program_id(0)
    a = pl.program_id(1)
    s = p * N_A + a
    parity = jax.lax.rem(s, 2)
    cur = parity * K
    nxt = (1 - parity) * K

    @pl.when(s == 0)
    def _first_gen():
        _gen_onehot(ridx_cur_ref, oh_ref, 0)

    _gen_onehot(ridx_nxt_ref, oh_ref, nxt)

    @pl.when(a == 0)
    def _zero():
        out_ref[...] = jnp.zeros_like(out_ref)

    out_ref[...] += jnp.dot(
        acc_ref[0], oh_ref[pl.ds(cur, K), :], preferred_element_type=jnp.float32
    )


@functools.partial(jax.jit, static_argnames=("interpret",))
def kernel(accumulator, interpret=False):
    n, c, a_dim, r_dim = accumulator.shape
    nc = n * c
    a_grid = a_dim // A_BLK
    ridx = jnp.asarray(_rho_index_table(OUT_H, OUT_W, NUMANGLE, NUMRHO))
    acc_p = jnp.pad(
        accumulator.reshape(nc, a_dim, r_dim), ((0, 0), (0, 0), (0, R_PAD - r_dim))
    )
    acc_g = (
        acc_p.reshape(nc, a_grid, A_BLK * R_PAD)
        .transpose(1, 0, 2)
        .astype(jnp.bfloat16)
    )

    def _nxt_map(p, a):
        roll = a == N_A - 1
        return (
            jnp.where(roll, 0, a + 1),
            0,
            jnp.where(roll, jnp.minimum(p + 1, N_P - 1), p),
        )

    out = pl.pallas_call(
        _idht_block,
        grid=(P // P_TILE, a_grid),
        in_specs=[
            pl.BlockSpec((1, A_BLK, P_TILE), lambda p, a: (a, 0, p)),
            pl.BlockSpec((1, A_BLK, P_TILE), _nxt_map),
            pl.BlockSpec((1, nc, K), lambda p, a: (a, 0, 0)),
        ],
        out_specs=pl.BlockSpec((nc, P_TILE), lambda p, a: (0, p)),
        out_shape=jax.ShapeDtypeStruct((nc, P), jnp.float32),
        scratch_shapes=[pltpu.VMEM((2 * K, P_TILE), jnp.bfloat16)],
        compiler_params=pltpu.CompilerParams(
            dimension_semantics=("parallel", "arbitrary"),
        ),
        interpret=interpret,
    )(ridx, ridx, acc_g)

    return out.reshape(n, c, OUT_H, OUT_W)


# speedup vs baseline: 1.0767x; 1.0767x over previous
import functools

import numpy as np
import jax
import jax.numpy as jnp
from jax.experimental import pallas as pl
from jax.experimental.pallas import tpu as pltpu

NUMANGLE = 180
A_PAD = 192
NUMRHO = 184
R_PAD = 192
OUT_H = 128
OUT_W = 128
P = OUT_H * OUT_W

P_TILE = 2048
A_BLK = 12
K = A_BLK * R_PAD
N_PAIR = A_PAD // (2 * A_BLK)
N_P = P // P_TILE


def _rho_index_table(H, W, numangle, numrho):
    irho = float(int(np.sqrt(H * H + W * W) + 1)) / float(numrho - 1)
    angles = np.arange(numangle).astype(np.float64) * (np.pi / numangle)
    cosi = np.cos(angles) / irho
    sini = np.sin(angles) / irho
    xs = (np.arange(W) - W // 2).astype(np.float64)
    ys = (np.arange(H) - H // 2).astype(np.float64)
    r = np.round(
        cosi[:, None, None] * xs[None, None, :] + sini[:, None, None] * ys[None, :, None]
    ).astype(np.int32) + numrho // 2
    invalid = (r < 0) | (r >= numrho)
    r[invalid] = numrho
    r = r.reshape(numangle, H * W)
    rp = np.full((A_PAD, H * W), numrho, np.int32)
    rp[:numangle] = r
    return rp.reshape(N_PAIR, 2 * A_BLK, H * W)


def _gen_onehot(ridx_ref, half, oh_ref):
    iota = jax.lax.broadcasted_iota(jnp.int32, (R_PAD, P_TILE), 0)
    for j in range(A_BLK):
        oh_ref[j * R_PAD : (j + 1) * R_PAD, :] = (
            iota == ridx_ref[0, half * A_BLK + j, :][None, :]
        ).astype(jnp.bfloat16)


def _idht_block(ridx_cur_ref, ridx_nxt_ref, acc_ref, out_ref, oh0_ref, oh1_ref):
    p = pl.program_id(0)
    t = pl.program_id(1)
    s = p * N_PAIR + t

    @pl.when(t == 0)
    def _zero():
        out_ref[...] = jnp.zeros_like(out_ref)

    @pl.when(s == 0)
    def _first_gen():
        _gen_onehot(ridx_cur_ref, 0, oh0_ref)
        _gen_onehot(ridx_cur_ref, 1, oh1_ref)

    out_ref[...] += jnp.dot(
        acc_ref[0, :, :K], oh0_ref[...], preferred_element_type=jnp.float32
    )
    _gen_onehot(ridx_nxt_ref, 0, oh0_ref)
    out_ref[...] += jnp.dot(
        acc_ref[0, :, K:], oh1_ref[...], preferred_element_type=jnp.float32
    )
    _gen_onehot(ridx_nxt_ref, 1, oh1_ref)


@functools.partial(jax.jit, static_argnames=("interpret",))
def kernel(accumulator, interpret=False):
    n, c, a_dim, r_dim = accumulator.shape
    nc = n * c
    ridx = jnp.asarray(_rho_index_table(OUT_H, OUT_W, NUMANGLE, NUMRHO))
    acc_p = jnp.pad(
        accumulator.reshape(nc, a_dim, r_dim),
        ((0, 0), (0, A_PAD - a_dim), (0, R_PAD - r_dim)),
    )
    acc_g = (
        acc_p.reshape(nc, N_PAIR, 2 * K).transpose(1, 0, 2).astype(jnp.bfloat16)
    )

    def _nxt_map(p, t):
        roll = t == N_PAIR - 1
        return (
            jnp.where(roll, 0, t + 1),
            0,
            jnp.where(roll, jnp.minimum(p + 1, N_P - 1), p),
        )

    out = pl.pallas_call(
        _idht_block,
        grid=(N_P, N_PAIR),
        in_specs=[
            pl.BlockSpec((1, 2 * A_BLK, P_TILE), lambda p, t: (t, 0, p)),
            pl.BlockSpec((1, 2 * A_BLK, P_TILE), _nxt_map),
            pl.BlockSpec((1, nc, 2 * K), lambda p, t: (t, 0, 0)),
        ],
        out_specs=pl.BlockSpec((nc, P_TILE), lambda p, t: (0, p)),
        out_shape=jax.ShapeDtypeStruct((nc, P), jnp.float32),
        scratch_shapes=[
            pltpu.VMEM((K, P_TILE), jnp.bfloat16),
            pltpu.VMEM((K, P_TILE), jnp.bfloat16),
        ],
        compiler_params=pltpu.CompilerParams(
            dimension_semantics=("parallel", "arbitrary"),
        ),
        interpret=interpret,
    )(ridx, ridx, acc_g)

    return out.reshape(n, c, OUT_H, OUT_W)


# speedup vs baseline: 1.1325x; 1.0519x over previous
import functools

import numpy as np
import ml_dtypes
import jax
import jax.numpy as jnp
from jax.experimental import pallas as pl
from jax.experimental.pallas import tpu as pltpu

NUMANGLE = 180
NUMRHO = 184
R_PAD = 192
OUT_H = 128
OUT_W = 128
P = OUT_H * OUT_W

P_TILE = 2048
A_BLK = 12
K = A_BLK * R_PAD
N_A = NUMANGLE // A_BLK
N_P = P // P_TILE

_BF16_ONE = np.uint16(0x3F80)


def _rho_index_table(H, W, numangle, numrho):
    irho = float(int(np.sqrt(H * H + W * W) + 1)) / float(numrho - 1)
    angles = np.arange(numangle).astype(np.float64) * (np.pi / numangle)
    cosi = np.cos(angles) / irho
    sini = np.sin(angles) / irho
    xs = (np.arange(W) - W // 2).astype(np.float64)
    ys = (np.arange(H) - H // 2).astype(np.float64)
    r = np.round(
        cosi[:, None, None] * xs[None, None, :] + sini[:, None, None] * ys[None, :, None]
    ).astype(np.int32) + numrho // 2
    invalid = (r < 0) | (r >= numrho)
    r[invalid] = numrho
    return r.reshape(numangle, H * W)


_TABLE_CACHE = {}


def _onehot_table():
    if "t" not in _TABLE_CACHE:
        r = _rho_index_table(OUT_H, OUT_W, NUMANGLE, NUMRHO)
        ks = np.arange(R_PAD, dtype=np.int32)
        out = np.zeros((N_A, K, P), np.uint16)
        for t in range(N_A):
            for j in range(A_BLK):
                m = r[t * A_BLK + j][None, :] == ks[:, None]
                blk = out[t, j * R_PAD : (j + 1) * R_PAD]
                blk[m] = _BF16_ONE
        _TABLE_CACHE["t"] = out.view(ml_dtypes.bfloat16)
    return _TABLE_CACHE["t"]


def _idht_block(oh_ref, acc_ref, out_ref):
    t = pl.program_id(1)

    @pl.when(t == 0)
    def _zero():
        out_ref[...] = jnp.zeros_like(out_ref)

    out_ref[...] += jnp.dot(
        acc_ref[0], oh_ref[0], preferred_element_type=jnp.float32
    )


@functools.partial(jax.jit, static_argnames=("interpret",))
def kernel(accumulator, interpret=False):
    n, c, a_dim, r_dim = accumulator.shape
    nc = n * c
    oh = jnp.asarray(_onehot_table())
    acc_p = jnp.pad(
        accumulator.reshape(nc, a_dim, r_dim), ((0, 0), (0, 0), (0, R_PAD - r_dim))
    )
    acc_g = (
        acc_p.reshape(nc, N_A, K).transpose(1, 0, 2).astype(jnp.bfloat16)
    )

    out = pl.pallas_call(
        _idht_block,
        grid=(N_P, N_A),
        in_specs=[
            pl.BlockSpec((1, K, P_TILE), lambda p, t: (t, 0, p)),
            pl.BlockSpec((1, nc, K), lambda p, t: (t, 0, 0)),
        ],
        out_specs=pl.BlockSpec((nc, P_TILE), lambda p, t: (0, p)),
        out_shape=jax.ShapeDtypeStruct((nc, P), jnp.float32),
        compiler_params=pltpu.CompilerParams(
            dimension_semantics=("parallel", "arbitrary"),
        ),
        interpret=interpret,
    )(oh, acc_g)

    return out.reshape(n, c, OUT_H, OUT_W)
